# baseline (speedup 1.0000x reference)
import functools

import jax
import jax.numpy as jnp
import numpy as np
from jax.experimental import pallas as pl
from jax.experimental.pallas import tpu as pltpu

N_B, H_S, W_S, CHANNELS = 2, 224, 224, 96
GROUP = 6
K = 3
PAD = 1
GC = CHANNELS // GROUP
EPS = 1e-6


def _matmul_body(y_ref, w_ref, b_ref, o_ref):
    o_ref[...] = (
        jnp.dot(y_ref[...], w_ref[...], preferred_element_type=jnp.float32)
        + b_ref[...]
    )


def _matmul_bias(y2d, w_t, b):
    m, c = y2d.shape
    blk = 1024
    grid = m // blk
    return pl.pallas_call(
        _matmul_body,
        grid=(grid,),
        in_specs=[
            pl.BlockSpec((blk, c), lambda i: (i, 0)),
            pl.BlockSpec((c, w_t.shape[1]), lambda i: (0, 0)),
            pl.BlockSpec((1, w_t.shape[1]), lambda i: (0, 0)),
        ],
        out_specs=pl.BlockSpec((blk, w_t.shape[1]), lambda i: (i, 0)),
        out_shape=jax.ShapeDtypeStruct((m, w_t.shape[1]), jnp.float32),
    )(y2d, w_t, b.reshape(1, -1))


def _bilinear(xp_g, py, px, Hp, Wp):
    y0 = jnp.floor(py)
    x0 = jnp.floor(px)
    wy = py - y0
    wx = px - x0
    y0i = y0.astype(jnp.int32)
    x0i = x0.astype(jnp.int32)
    n = xp_g.shape[0]
    g = xp_g.shape[3]
    bb = jnp.arange(n)[:, None, None, None]
    gg = jnp.arange(g)[None, None, None, :]

    def gat(yi, xi):
        valid = ((yi >= 0) & (yi < Hp) & (xi >= 0) & (xi < Wp)).astype(xp_g.dtype)
        yc = jnp.clip(yi, 0, Hp - 1)
        xc = jnp.clip(xi, 0, Wp - 1)
        v = xp_g[bb, yc, xc, gg]
        return v * valid[..., None]

    v00 = gat(y0i, x0i)
    v01 = gat(y0i, x0i + 1)
    v10 = gat(y0i + 1, x0i)
    v11 = gat(y0i + 1, x0i + 1)
    w00 = ((1 - wy) * (1 - wx))[..., None]
    w01 = ((1 - wy) * wx)[..., None]
    w10 = (wy * (1 - wx))[..., None]
    w11 = (wy * wx)[..., None]
    return v00 * w00 + v01 * w01 + v10 * w10 + v11 * w11


def kernel(x, depth, W_in, b_in, W_dw, b_dw, ln_g, ln_b, W_off, b_off,
           W_mask, b_mask, W_out, b_out):
    N, H, W, C = x.shape
    x_proj = _matmul_bias(x.reshape(-1, C), W_in.T, b_in).reshape(N, H, W, C)
    x1 = jnp.transpose(x, (0, 3, 1, 2))
    x1 = jax.lax.conv_general_dilated(
        x1, W_dw, window_strides=(1, 1),
        padding=((PAD, PAD), (PAD, PAD)), feature_group_count=C,
        dimension_numbers=("NCHW", "OIHW", "NCHW"))
    x1 = x1 + b_dw[None, :, None, None]
    x1 = jnp.transpose(x1, (0, 2, 3, 1))
    mu = jnp.mean(x1, axis=-1, keepdims=True)
    var = jnp.var(x1, axis=-1, keepdims=True)
    x1 = (x1 - mu) / jnp.sqrt(var + EPS) * ln_g + ln_b
    x1 = jax.nn.gelu(x1, approximate=False)
    offset = x1 @ W_off.T + b_off
    mask_logits = x1 @ W_mask.T + b_mask
    mask = jax.nn.softmax(mask_logits.reshape(N, H, W, GROUP, K * K), axis=-1)
    xp = jnp.pad(x_proj, ((0, 0), (PAD, PAD), (PAD, PAD), (0, 0)))
    Hp, Wp = H + 2 * PAD, W + 2 * PAD
    xp_g = xp.reshape(N, Hp, Wp, GROUP, GC)
    off = offset.reshape(N, H, W, GROUP, K * K, 2)
    pts = np.arange(K) - (K - 1) // 2
    dy, dx = np.meshgrid(pts, pts, indexing="ij")
    dy = dy.reshape(-1)
    dx = dx.reshape(-1)
    base_y = (jnp.arange(H) + PAD).astype(x.dtype)[None, :, None, None]
    base_x = (jnp.arange(W) + PAD).astype(x.dtype)[None, None, :, None]
    acc = jnp.zeros((N, H, W, GROUP, GC), dtype=x.dtype)
    for k in range(K * K):
        py = base_y + float(dy[k]) + off[..., k, 0]
        px = base_x + float(dx[k]) + off[..., k, 1]
        samp = _bilinear(xp_g, py, px, Hp, Wp)
        acc = acc + samp * mask[..., k][..., None]
    y = acc.reshape(N, H, W, C)
    out = _matmul_bias(y.reshape(-1, C), W_out.T, b_out).reshape(N, H, W, C)
    return (out, depth)


# speedup vs baseline: 20.1727x; 20.1727x over previous
import functools

import jax
import jax.numpy as jnp
from jax import lax
from jax.experimental import pallas as pl
from jax.experimental.pallas import tpu as pltpu
from jax.experimental.pallas import tpu_sc as plsc

N_B, H_S, W_S, C_CH = 2, 224, 224, 96
GROUP = 6
GC = C_CH // GROUP
KK = 9
PAD = 1
EPS = 1e-6
HP = H_S + 2 * PAD
NPIX = N_B * H_S * W_S
NPAIR = NPIX * GROUP
NTAB = N_B * HP * HP * GROUP

R_BLK = 16
NBLK = H_S // R_BLK
M_BLK = R_BLK * W_S

NW = 32
PIX_W = NPIX // NW
CHUNK_PIX = 16
NCHUNK = PIX_W // CHUNK_PIX
IDX_ROWS = (CHUNK_PIX * 216) // 128


def _matmul_body(y_ref, w_ref, b_ref, o_ref):
    o_ref[...] = (
        jnp.dot(y_ref[...], w_ref[...], preferred_element_type=jnp.float32)
        + b_ref[...]
    )


def _matmul_bias(y2d, w_t, b):
    m, c = y2d.shape
    blk = 1024
    return pl.pallas_call(
        _matmul_body,
        grid=(m // blk,),
        in_specs=[
            pl.BlockSpec((blk, c), lambda i: (i, 0)),
            pl.BlockSpec((c, w_t.shape[1]), lambda i: (0, 0)),
            pl.BlockSpec((1, w_t.shape[1]), lambda i: (0, 0)),
        ],
        out_specs=pl.BlockSpec((blk, w_t.shape[1]), lambda i: (i, 0)),
        out_shape=jax.ShapeDtypeStruct((m, w_t.shape[1]), jnp.float32),
    )(y2d, w_t, b.reshape(1, -1))


def _erf(z):
    s = jnp.sign(z)
    za = jnp.abs(z)
    t = 1.0 / (1.0 + 0.3275911 * za)
    poly = t * (0.254829592 + t * (-0.284496736 + t * (1.421413741
               + t * (-1.453152027 + t * 1.061405429))))
    return s * (1.0 - poly * jnp.exp(-za * za))


def _prep_body(xm_ref, xc_ref, xp_ref, wdw_ref, bdw_ref, lng_ref, lnb_ref,
               wofft_ref, boff_ref, wmaskt_ref, bmask_ref, idx_ref, wts_ref):
    n = pl.program_id(0)
    i = pl.program_id(1)
    f32 = jnp.float32

    top = xm_ref[0, R_BLK - 1:R_BLK]
    bot = xp_ref[0, 0:1]
    xs = jnp.concatenate([top, xc_ref[0], bot], axis=0)
    ri = lax.broadcasted_iota(jnp.int32, (R_BLK + 2, 1, 1), 0)
    ok = ((ri != 0) | (i > 0)) & ((ri != R_BLK + 1) | (i < NBLK - 1))
    xs = xs * ok.astype(f32)

    acc = jnp.zeros((R_BLK, W_S, C_CH), f32)
    for ky in range(3):
        rows = xs[ky:ky + R_BLK]
        for kx in range(3):
            if kx == 0:
                sh = jnp.concatenate(
                    [jnp.zeros((R_BLK, 1, C_CH), f32), rows[:, :W_S - 1]], axis=1)
            elif kx == 1:
                sh = rows
            else:
                sh = jnp.concatenate(
                    [rows[:, 1:], jnp.zeros((R_BLK, 1, C_CH), f32)], axis=1)
            acc = acc + sh * wdw_ref[ky * 3 + kx]
    x1 = acc.reshape(M_BLK, C_CH) + bdw_ref[0]
    mu = jnp.mean(x1, axis=-1, keepdims=True)
    var = jnp.mean((x1 - mu) ** 2, axis=-1, keepdims=True)
    x1 = (x1 - mu) / jnp.sqrt(var + EPS) * lng_ref[0] + lnb_ref[0]
    x1 = x1 * 0.5 * (1.0 + _erf(x1 * 0.7071067811865476))

    offm = jnp.dot(x1, wofft_ref[...], preferred_element_type=f32) + boff_ref[0]
    ml = jnp.dot(x1, wmaskt_ref[...], preferred_element_type=f32) + bmask_ref[0]
    mx = jnp.max(ml, axis=-1, keepdims=True)
    e = jnp.exp(ml - mx)
    r54 = lax.broadcasted_iota(jnp.int32, (54, 54), 0)
    c54 = lax.broadcasted_iota(jnp.int32, (54, 54), 1)
    seg = ((r54 // KK) == (c54 // KK)).astype(f32)
    gs = jnp.dot(e, seg, preferred_element_type=f32)
    msk = e / gs

    mi = lax.broadcasted_iota(jnp.int32, (M_BLK, 1), 0)
    yb = (mi // W_S) + i * R_BLK + PAD
    xb = (mi % W_S) + PAD
    kk = lax.broadcasted_iota(jnp.int32, (1, 54), 1) % KK
    g54 = lax.broadcasted_iota(jnp.int32, (1, 54), 1) // KK
    dy = kk // 3 - 1
    dx = kk % 3 - 1
    py = yb.astype(f32) + dy.astype(f32) + offm[:, :54]
    px = xb.astype(f32) + dx.astype(f32) + offm[:, 54:]
    y0f = jnp.floor(py)
    x0f = jnp.floor(px)
    wy = py - y0f
    wx = px - x0f
    y0 = y0f.astype(jnp.int32)
    x0 = x0f.astype(jnp.int32)

    def cidx(iy, ix):
        v = ((iy >= 0) & (iy < HP) & (ix >= 0) & (ix < HP)).astype(f32)
        iyc = jnp.clip(iy, 0, HP - 1)
        ixc = jnp.clip(ix, 0, HP - 1)
        idx = ((n * HP + iyc) * HP + ixc) * GROUP + g54
        return idx, v

    i00, v00 = cidx(y0, x0)
    i01, v01 = cidx(y0, x0 + 1)
    i10, v10 = cidx(y0 + 1, x0)
    i11, v11 = cidx(y0 + 1, x0 + 1)
    wy1 = 1.0 - wy
    wx1 = 1.0 - wx
    idx_ref[:, 0:54] = i00
    idx_ref[:, 54:108] = i01
    idx_ref[:, 108:162] = i10
    idx_ref[:, 162:216] = i11
    wts_ref[:, 0:54] = wy1 * wx1 * msk * v00
    wts_ref[:, 54:108] = wy1 * wx * msk * v01
    wts_ref[:, 108:162] = wy * wx1 * msk * v10
    wts_ref[:, 162:216] = wy * wx * msk * v11


def _prep(x, wdw9, b_dw, ln_g, ln_b, wofft, boffr, wmaskt, b_mask):
    xb = lambda d: pl.BlockSpec(
        (1, R_BLK, W_S, C_CH),
        lambda n, i, d=d: (n, jnp.clip(i + d, 0, NBLK - 1), 0, 0))
    full = lambda a: pl.BlockSpec(a.shape, lambda n, i: (0,) * a.ndim)
    outs = [
        jax.ShapeDtypeStruct((NPIX, 216), jnp.int32),
        jax.ShapeDtypeStruct((NPIX, 216), jnp.float32),
    ]
    ospec = pl.BlockSpec((M_BLK, 216), lambda n, i: (n * NBLK + i, 0))
    return pl.pallas_call(
        _prep_body,
        grid=(N_B, NBLK),
        in_specs=[xb(-1), xb(0), xb(1), full(wdw9), full(b_dw), full(ln_g),
                  full(ln_b), full(wofft), full(boffr), full(wmaskt),
                  full(b_mask)],
        out_specs=[ospec, ospec],
        out_shape=outs,
    )(x, x, x, wdw9, b_dw, ln_g, ln_b, wofft, boffr, wmaskt, b_mask)


def _sc_sample(table, idx2, wts1):
    mesh = plsc.VectorSubcoreMesh(core_axis_name="c", subcore_axis_name="s")

    @functools.partial(
        pl.kernel, mesh=mesh,
        compiler_params=pltpu.CompilerParams(use_tc_tiling_on_sc=False),
        out_type=jax.ShapeDtypeStruct((NPAIR, GC), jnp.float32),
        scratch_types=[
            pltpu.VMEM((CHUNK_PIX * 216,), jnp.int32),
            pltpu.VMEM((CHUNK_PIX * 216 + 16,), jnp.float32),
            pltpu.VMEM((CHUNK_PIX * 216, GC), jnp.float32),
            pltpu.VMEM((CHUNK_PIX * GROUP, GC), jnp.float32),
            pltpu.SemaphoreType.DMA,
        ],
    )
    def body(table_h, idx_h, wts_h, y_h, idx_v, wts_v, rows_v, out_v, sem):
        wid = lax.axis_index("s") * 2 + lax.axis_index("c")

        def chunk(t, carry):
            gchunk = wid * NCHUNK + t
            pix0 = gchunk * CHUNK_PIX
            pltpu.sync_copy(idx_h.at[pl.ds(pix0 * 216, CHUNK_PIX * 216)],
                            idx_v)
            pltpu.sync_copy(wts_h.at[pl.ds(pix0 * 216, CHUNK_PIX * 216)],
                            wts_v.at[pl.ds(0, CHUNK_PIX * 216)])
            handles = [
                pltpu.async_copy(table_h.at[idx_v.at[pl.ds(j * 128, 128)]],
                                 rows_v.at[pl.ds(j * 128, 128)], sem)
                for j in range(IDX_ROWS)
            ]
            for h in handles:
                h.wait()

            def per_pix(p, c2):
                base = p * 216
                accs = [jnp.zeros((GC,), jnp.float32) for _ in range(GROUP)]
                for c16 in range(14):
                    w16 = wts_v[pl.ds(base + c16 * 16, 16)]
                    nrow = 16 if c16 < 13 else 8
                    for t in range(nrow):
                        j = c16 * 16 + t
                        g = (j % 54) // KK
                        wv = jnp.full((GC,), w16[t], jnp.float32)
                        accs[g] = accs[g] + wv * rows_v[base + j, :]
                for g in range(GROUP):
                    out_v[p * GROUP + g, :] = accs[g]
                return c2

            lax.fori_loop(0, CHUNK_PIX, per_pix, 0)
            pltpu.sync_copy(out_v,
                            y_h.at[pl.ds(pix0 * GROUP, CHUNK_PIX * GROUP)])
            return carry

        lax.fori_loop(0, NCHUNK, chunk, 0)

    return body(table, idx2, wts1)


def kernel(x, depth, W_in, b_in, W_dw, b_dw, ln_g, ln_b, W_off, b_off,
           W_mask, b_mask, W_out, b_out):
    f32 = jnp.float32
    x2d = x.reshape(NPIX, C_CH)
    x_proj = _matmul_bias(x2d, W_in.T, b_in)
    table = jnp.pad(
        x_proj.reshape(N_B, H_S, W_S, C_CH),
        ((0, 0), (PAD, PAD), (PAD, PAD), (0, 0))).reshape(NTAB, GC)

    wdw9 = W_dw.reshape(C_CH, KK).T
    woy = W_off[0::2]
    wox = W_off[1::2]
    wofft = jnp.concatenate([woy, wox], axis=0).T
    boffr = jnp.concatenate([b_off[0::2], b_off[1::2]]).reshape(1, -1)
    idx, wts = _prep(x, wdw9, b_dw.reshape(1, -1), ln_g.reshape(1, -1),
                     ln_b.reshape(1, -1), wofft, boffr, W_mask.T,
                     b_mask.reshape(1, -1))
    idx2 = idx.reshape(-1)
    wts1 = wts.reshape(-1)
    y = _sc_sample(table, idx2, wts1)
    out = _matmul_bias(y.reshape(NPIX, C_CH), W_out.T, b_out)
    return (out.reshape(N_B, H_S, W_S, C_CH).astype(f32), depth)


# speedup vs baseline: 27.1322x; 1.3450x over previous
import functools

import jax
import jax.numpy as jnp
from jax import lax
from jax.experimental import pallas as pl
from jax.experimental.pallas import tpu as pltpu
from jax.experimental.pallas import tpu_sc as plsc

N_B, H_S, W_S, C_CH = 2, 224, 224, 96
GROUP = 6
GC = C_CH // GROUP
KK = 9
PAD = 1
EPS = 1e-6
HP = H_S + 2 * PAD
NPIX = N_B * H_S * W_S
NPAIR = NPIX * GROUP
NTAB = N_B * HP * HP * GROUP

R_BLK = 16
NBLK = H_S // R_BLK
M_BLK = R_BLK * W_S

NW = 32
PIX_W = NPIX // NW
CHUNK_PIX = 16
NCHUNK = PIX_W // CHUNK_PIX
IDX_ROWS = (CHUNK_PIX * 216) // 128


def _matmul_body(y_ref, w_ref, b_ref, o_ref):
    o_ref[...] = (
        jnp.dot(y_ref[...], w_ref[...], preferred_element_type=jnp.float32)
        + b_ref[...]
    )


def _matmul_bias(y2d, w_t, b):
    m, c = y2d.shape
    blk = 1024
    return pl.pallas_call(
        _matmul_body,
        grid=(m // blk,),
        in_specs=[
            pl.BlockSpec((blk, c), lambda i: (i, 0)),
            pl.BlockSpec((c, w_t.shape[1]), lambda i: (0, 0)),
            pl.BlockSpec((1, w_t.shape[1]), lambda i: (0, 0)),
        ],
        out_specs=pl.BlockSpec((blk, w_t.shape[1]), lambda i: (i, 0)),
        out_shape=jax.ShapeDtypeStruct((m, w_t.shape[1]), jnp.float32),
    )(y2d, w_t, b.reshape(1, -1))


def _erf(z):
    s = jnp.sign(z)
    za = jnp.abs(z)
    t = 1.0 / (1.0 + 0.3275911 * za)
    poly = t * (0.254829592 + t * (-0.284496736 + t * (1.421413741
               + t * (-1.453152027 + t * 1.061405429))))
    return s * (1.0 - poly * jnp.exp(-za * za))


def _prep_body(xm_ref, xc_ref, xp_ref, wdw_ref, bdw_ref, lng_ref, lnb_ref,
               wofft_ref, boff_ref, wmaskt_ref, bmask_ref, idx_ref, wts_ref):
    n = pl.program_id(0)
    i = pl.program_id(1)
    f32 = jnp.float32

    top = xm_ref[0, R_BLK - 1:R_BLK]
    bot = xp_ref[0, 0:1]
    xs = jnp.concatenate([top, xc_ref[0], bot], axis=0)
    ri = lax.broadcasted_iota(jnp.int32, (R_BLK + 2, 1, 1), 0)
    ok = ((ri != 0) | (i > 0)) & ((ri != R_BLK + 1) | (i < NBLK - 1))
    xs = xs * ok.astype(f32)

    acc = jnp.zeros((R_BLK, W_S, C_CH), f32)
    for ky in range(3):
        rows = xs[ky:ky + R_BLK]
        for kx in range(3):
            if kx == 0:
                sh = jnp.concatenate(
                    [jnp.zeros((R_BLK, 1, C_CH), f32), rows[:, :W_S - 1]], axis=1)
            elif kx == 1:
                sh = rows
            else:
                sh = jnp.concatenate(
                    [rows[:, 1:], jnp.zeros((R_BLK, 1, C_CH), f32)], axis=1)
            acc = acc + sh * wdw_ref[ky * 3 + kx]
    x1 = acc.reshape(M_BLK, C_CH) + bdw_ref[0]
    mu = jnp.mean(x1, axis=-1, keepdims=True)
    var = jnp.mean((x1 - mu) ** 2, axis=-1, keepdims=True)
    x1 = (x1 - mu) / jnp.sqrt(var + EPS) * lng_ref[0] + lnb_ref[0]
    x1 = x1 * 0.5 * (1.0 + _erf(x1 * 0.7071067811865476))

    offm = jnp.dot(x1, wofft_ref[...], preferred_element_type=f32) + boff_ref[0]
    ml = jnp.dot(x1, wmaskt_ref[...], preferred_element_type=f32) + bmask_ref[0]
    mx = jnp.max(ml, axis=-1, keepdims=True)
    e = jnp.exp(ml - mx)
    r54 = lax.broadcasted_iota(jnp.int32, (54, 54), 0)
    c54 = lax.broadcasted_iota(jnp.int32, (54, 54), 1)
    seg = ((r54 // KK) == (c54 // KK)).astype(f32)
    gs = jnp.dot(e, seg, preferred_element_type=f32)
    msk = e / gs

    mi = lax.broadcasted_iota(jnp.int32, (M_BLK, 1), 0)
    yb = (mi // W_S) + i * R_BLK + PAD
    xb = (mi % W_S) + PAD
    kk = lax.broadcasted_iota(jnp.int32, (1, 54), 1) % KK
    g54 = lax.broadcasted_iota(jnp.int32, (1, 54), 1) // KK
    dy = kk // 3 - 1
    dx = kk % 3 - 1
    py = yb.astype(f32) + dy.astype(f32) + offm[:, :54]
    px = xb.astype(f32) + dx.astype(f32) + offm[:, 54:]
    y0f = jnp.floor(py)
    x0f = jnp.floor(px)
    wy = py - y0f
    wx = px - x0f
    y0 = y0f.astype(jnp.int32)
    x0 = x0f.astype(jnp.int32)

    def cidx(iy, ix):
        v = ((iy >= 0) & (iy < HP) & (ix >= 0) & (ix < HP)).astype(f32)
        iyc = jnp.clip(iy, 0, HP - 1)
        ixc = jnp.clip(ix, 0, HP - 1)
        idx = ((n * HP + iyc) * HP + ixc) * GROUP + g54
        return idx, v

    i00, v00 = cidx(y0, x0)
    i01, v01 = cidx(y0, x0 + 1)
    i10, v10 = cidx(y0 + 1, x0)
    i11, v11 = cidx(y0 + 1, x0 + 1)
    wy1 = 1.0 - wy
    wx1 = 1.0 - wx
    idx_ref[:, 0:54] = i00
    idx_ref[:, 54:108] = i01
    idx_ref[:, 108:162] = i10
    idx_ref[:, 162:216] = i11
    wts_ref[:, 0:54] = wy1 * wx1 * msk * v00
    wts_ref[:, 54:108] = wy1 * wx * msk * v01
    wts_ref[:, 108:162] = wy * wx1 * msk * v10
    wts_ref[:, 162:216] = wy * wx * msk * v11


def _prep(x, wdw9, b_dw, ln_g, ln_b, wofft, boffr, wmaskt, b_mask):
    xb = lambda d: pl.BlockSpec(
        (1, R_BLK, W_S, C_CH),
        lambda n, i, d=d: (n, jnp.clip(i + d, 0, NBLK - 1), 0, 0))
    full = lambda a: pl.BlockSpec(a.shape, lambda n, i: (0,) * a.ndim)
    outs = [
        jax.ShapeDtypeStruct((NPIX, 216), jnp.int32),
        jax.ShapeDtypeStruct((NPIX, 216), jnp.float32),
    ]
    ospec = pl.BlockSpec((M_BLK, 216), lambda n, i: (n * NBLK + i, 0))
    return pl.pallas_call(
        _prep_body,
        grid=(N_B, NBLK),
        in_specs=[xb(-1), xb(0), xb(1), full(wdw9), full(b_dw), full(ln_g),
                  full(ln_b), full(wofft), full(boffr), full(wmaskt),
                  full(b_mask)],
        out_specs=[ospec, ospec],
        out_shape=outs,
    )(x, x, x, wdw9, b_dw, ln_g, ln_b, wofft, boffr, wmaskt, b_mask)


def _sc_sample(table, idx2, wts1):
    mesh = plsc.VectorSubcoreMesh(core_axis_name="c", subcore_axis_name="s")

    @functools.partial(
        pl.kernel, mesh=mesh,
        compiler_params=pltpu.CompilerParams(use_tc_tiling_on_sc=False),
        out_type=jax.ShapeDtypeStruct((NPAIR, GC), jnp.float32),
        scratch_types=[
            pltpu.VMEM((2, CHUNK_PIX * 216), jnp.int32),
            pltpu.VMEM((2, CHUNK_PIX * 216 + 16), jnp.float32),
            pltpu.VMEM((2 * CHUNK_PIX * 216, GC), jnp.float32),
            pltpu.VMEM((CHUNK_PIX * GROUP, GC), jnp.float32),
            pltpu.SemaphoreType.DMA,
            pltpu.SemaphoreType.DMA,
            pltpu.SemaphoreType.DMA,
            pltpu.SemaphoreType.DMA,
        ],
    )
    def body(table_h, idx_h, wts_h, y_h, idx_v, wts_v, rows_v, out_v,
             sg0, sg1, si0, si1):
        wid = lax.axis_index("s") * 2 + lax.axis_index("c")
        sg = [sg0, sg1]
        si = [si0, si1]
        NV = CHUNK_PIX * 216

        def load_idx(t, b, sem):
            tc = jnp.minimum(t, NCHUNK - 1)
            pix0 = (wid * NCHUNK + tc) * CHUNK_PIX
            return pltpu.async_copy(idx_h.at[pl.ds(pix0 * 216, NV)],
                                    idx_v.at[b], sem)

        def load_wts(t, b, sem):
            tc = jnp.minimum(t, NCHUNK - 1)
            pix0 = (wid * NCHUNK + tc) * CHUNK_PIX
            return pltpu.async_copy(wts_h.at[pl.ds(pix0 * 216, NV)],
                                    wts_v.at[b, pl.ds(0, NV)], sem)

        def fire(b):
            return [
                pltpu.async_copy(
                    table_h.at[idx_v.at[b, pl.ds(j * 128, 128)]],
                    rows_v.at[pl.ds(b * NV + j * 128, 128)], sg[b])
                for j in range(IDX_ROWS)
            ]

        def compute(t, b):
            pix0 = (wid * NCHUNK + t) * CHUNK_PIX

            def per_pix(p, c2):
                base = p * 216
                accs = [jnp.zeros((GC,), jnp.float32) for _ in range(GROUP)]
                for c16 in range(14):
                    w16 = wts_v[b, pl.ds(base + c16 * 16, 16)]
                    nrow = 16 if c16 < 13 else 8
                    for tt in range(nrow):
                        j = c16 * 16 + tt
                        g = (j % 54) // KK
                        wv = jnp.full((GC,), w16[tt], jnp.float32)
                        accs[g] = accs[g] + wv * rows_v[b * NV + base + j, :]
                for g in range(GROUP):
                    out_v[p * GROUP + g, :] = accs[g]
                return c2

            lax.fori_loop(0, CHUNK_PIX, per_pix, 0)
            pltpu.sync_copy(out_v,
                            y_h.at[pl.ds(pix0 * GROUP, CHUNK_PIX * GROUP)])

        load_idx(0, 0, si[0]).wait()
        load_wts(0, 0, si[0]).wait()
        fire(0)
        load_idx(1, 1, si[1]).wait()
        load_wts(1, 1, si[1]).wait()

        def step(m, carry):
            for b in (0, 1):
                c = 2 * m + b

                @pl.when(c + 1 < NCHUNK)
                def _():
                    fire(1 - b)
                for j in range(IDX_ROWS):
                    pltpu.make_async_copy(
                        table_h.at[idx_v.at[b, pl.ds(j * 128, 128)]],
                        rows_v.at[pl.ds(b * NV + j * 128, 128)],
                        sg[b]).wait()
                p1 = load_idx(c + 2, b, si[b])
                compute(c, b)
                p2 = load_wts(c + 2, b, si[b])
                p1.wait()
                p2.wait()
            return carry

        lax.fori_loop(0, NCHUNK // 2, step, 0)

    return body(table, idx2, wts1)


def kernel(x, depth, W_in, b_in, W_dw, b_dw, ln_g, ln_b, W_off, b_off,
           W_mask, b_mask, W_out, b_out):
    f32 = jnp.float32
    x2d = x.reshape(NPIX, C_CH)
    x_proj = _matmul_bias(x2d, W_in.T, b_in)
    table = jnp.pad(
        x_proj.reshape(N_B, H_S, W_S, C_CH),
        ((0, 0), (PAD, PAD), (PAD, PAD), (0, 0))).reshape(NTAB, GC)

    wdw9 = W_dw.reshape(C_CH, KK).T
    woy = W_off[0::2]
    wox = W_off[1::2]
    wofft = jnp.concatenate([woy, wox], axis=0).T
    boffr = jnp.concatenate([b_off[0::2], b_off[1::2]]).reshape(1, -1)
    idx, wts = _prep(x, wdw9, b_dw.reshape(1, -1), ln_g.reshape(1, -1),
                     ln_b.reshape(1, -1), wofft, boffr, W_mask.T,
                     b_mask.reshape(1, -1))
    idx2 = idx.reshape(-1)
    wts1 = wts.reshape(-1)
    y = _sc_sample(table, idx2, wts1)
    out = _matmul_bias(y.reshape(NPIX, C_CH), W_out.T, b_out)
    return (out.reshape(N_B, H_S, W_S, C_CH).astype(f32), depth)


# speedup vs baseline: 30.4778x; 1.1233x over previous
import functools

import jax
import jax.numpy as jnp
from jax import lax
from jax.experimental import pallas as pl
from jax.experimental.pallas import tpu as pltpu
from jax.experimental.pallas import tpu_sc as plsc

N_B, H_S, W_S, C_CH = 2, 224, 224, 96
GROUP = 6
GC = C_CH // GROUP
KK = 9
PAD = 1
EPS = 1e-6
HP = H_S + 2 * PAD
NPIX = N_B * H_S * W_S
NPAIR = NPIX * GROUP
NTAB = N_B * HP * HP * GROUP

R_BLK = 16
NBLK = H_S // R_BLK
M_BLK = R_BLK * W_S

NW = 32
NPIX2 = NPIX // N_B
NPAIR2 = NPIX2 * GROUP
NTAB2 = HP * HP * GROUP
CHUNK_PIX = 16
NCHUNK = NPIX2 // NW // CHUNK_PIX
IDX_ROWS = (CHUNK_PIX * 216) // 128


def _matmul_body(y_ref, w_ref, b_ref, o_ref):
    o_ref[...] = (
        jnp.dot(y_ref[...], w_ref[...], preferred_element_type=jnp.float32)
        + b_ref[...]
    )


def _matmul_bias(y2d, w_t, b):
    m, c = y2d.shape
    blk = 1024
    return pl.pallas_call(
        _matmul_body,
        grid=(m // blk,),
        in_specs=[
            pl.BlockSpec((blk, c), lambda i: (i, 0)),
            pl.BlockSpec((c, w_t.shape[1]), lambda i: (0, 0)),
            pl.BlockSpec((1, w_t.shape[1]), lambda i: (0, 0)),
        ],
        out_specs=pl.BlockSpec((blk, w_t.shape[1]), lambda i: (i, 0)),
        out_shape=jax.ShapeDtypeStruct((m, w_t.shape[1]), jnp.float32),
    )(y2d, w_t, b.reshape(1, -1))


def _erf(z):
    s = jnp.sign(z)
    za = jnp.abs(z)
    t = 1.0 / (1.0 + 0.3275911 * za)
    poly = t * (0.254829592 + t * (-0.284496736 + t * (1.421413741
               + t * (-1.453152027 + t * 1.061405429))))
    return s * (1.0 - poly * jnp.exp(-za * za))


def _prep_body(xm_ref, xc_ref, xp_ref, wdw_ref, bdw_ref, lng_ref, lnb_ref,
               wofft_ref, boff_ref, wmaskt_ref, bmask_ref, idx_ref, wts_ref):
    n = pl.program_id(0)
    i = pl.program_id(1)
    f32 = jnp.float32

    top = xm_ref[0, R_BLK - 1:R_BLK]
    bot = xp_ref[0, 0:1]
    xs = jnp.concatenate([top, xc_ref[0], bot], axis=0)
    ri = lax.broadcasted_iota(jnp.int32, (R_BLK + 2, 1, 1), 0)
    ok = ((ri != 0) | (i > 0)) & ((ri != R_BLK + 1) | (i < NBLK - 1))
    xs = xs * ok.astype(f32)

    acc = jnp.zeros((R_BLK, W_S, C_CH), f32)
    for ky in range(3):
        rows = xs[ky:ky + R_BLK]
        for kx in range(3):
            if kx == 0:
                sh = jnp.concatenate(
                    [jnp.zeros((R_BLK, 1, C_CH), f32), rows[:, :W_S - 1]], axis=1)
            elif kx == 1:
                sh = rows
            else:
                sh = jnp.concatenate(
                    [rows[:, 1:], jnp.zeros((R_BLK, 1, C_CH), f32)], axis=1)
            acc = acc + sh * wdw_ref[ky * 3 + kx]
    x1 = acc.reshape(M_BLK, C_CH) + bdw_ref[0]
    mu = jnp.mean(x1, axis=-1, keepdims=True)
    var = jnp.mean((x1 - mu) ** 2, axis=-1, keepdims=True)
    x1 = (x1 - mu) / jnp.sqrt(var + EPS) * lng_ref[0] + lnb_ref[0]
    x1 = x1 * 0.5 * (1.0 + _erf(x1 * 0.7071067811865476))

    offm = jnp.dot(x1, wofft_ref[...], preferred_element_type=f32) + boff_ref[0]
    ml = jnp.dot(x1, wmaskt_ref[...], preferred_element_type=f32) + bmask_ref[0]
    mx = jnp.max(ml, axis=-1, keepdims=True)
    e = jnp.exp(ml - mx)
    r54 = lax.broadcasted_iota(jnp.int32, (54, 54), 0)
    c54 = lax.broadcasted_iota(jnp.int32, (54, 54), 1)
    seg = ((r54 // KK) == (c54 // KK)).astype(f32)
    gs = jnp.dot(e, seg, preferred_element_type=f32)
    msk = e / gs

    mi = lax.broadcasted_iota(jnp.int32, (M_BLK, 1), 0)
    yb = (mi // W_S) + i * R_BLK + PAD
    xb = (mi % W_S) + PAD
    kk = lax.broadcasted_iota(jnp.int32, (1, 54), 1) % KK
    g54 = lax.broadcasted_iota(jnp.int32, (1, 54), 1) // KK
    dy = kk // 3 - 1
    dx = kk % 3 - 1
    py = yb.astype(f32) + dy.astype(f32) + offm[:, :54]
    px = xb.astype(f32) + dx.astype(f32) + offm[:, 54:]
    y0f = jnp.floor(py)
    x0f = jnp.floor(px)
    wy = py - y0f
    wx = px - x0f
    y0 = y0f.astype(jnp.int32)
    x0 = x0f.astype(jnp.int32)

    def cidx(iy, ix):
        v = ((iy >= 0) & (iy < HP) & (ix >= 0) & (ix < HP)).astype(f32)
        iyc = jnp.clip(iy, 0, HP - 1)
        ixc = jnp.clip(ix, 0, HP - 1)
        idx = ((n * HP + iyc) * HP + ixc) * GROUP + g54
        return idx, v

    i00, v00 = cidx(y0, x0)
    i01, v01 = cidx(y0, x0 + 1)
    i10, v10 = cidx(y0 + 1, x0)
    i11, v11 = cidx(y0 + 1, x0 + 1)
    wy1 = 1.0 - wy
    wx1 = 1.0 - wx
    idx_ref[:, 0:54] = i00
    idx_ref[:, 54:108] = i01
    idx_ref[:, 108:162] = i10
    idx_ref[:, 162:216] = i11
    wts_ref[:, 0:54] = wy1 * wx1 * msk * v00
    wts_ref[:, 54:108] = wy1 * wx * msk * v01
    wts_ref[:, 108:162] = wy * wx1 * msk * v10
    wts_ref[:, 162:216] = wy * wx * msk * v11


def _prep(x, wdw9, b_dw, ln_g, ln_b, wofft, boffr, wmaskt, b_mask):
    xb = lambda d: pl.BlockSpec(
        (1, R_BLK, W_S, C_CH),
        lambda n, i, d=d: (n, jnp.clip(i + d, 0, NBLK - 1), 0, 0))
    full = lambda a: pl.BlockSpec(a.shape, lambda n, i: (0,) * a.ndim)
    outs = [
        jax.ShapeDtypeStruct((NPIX2, 216), jnp.int32),
        jax.ShapeDtypeStruct((NPIX2, 216), jnp.float32),
    ]
    ospec = pl.BlockSpec((M_BLK, 216), lambda n, i: (n * NBLK + i, 0))
    return pl.pallas_call(
        _prep_body,
        grid=(1, NBLK),
        in_specs=[xb(-1), xb(0), xb(1), full(wdw9), full(b_dw), full(ln_g),
                  full(ln_b), full(wofft), full(boffr), full(wmaskt),
                  full(b_mask)],
        out_specs=[ospec, ospec],
        out_shape=outs,
    )(x, x, x, wdw9, b_dw, ln_g, ln_b, wofft, boffr, wmaskt, b_mask)


def _sc_sample(table, idx2, wts1):
    mesh = plsc.VectorSubcoreMesh(core_axis_name="c", subcore_axis_name="s")

    @functools.partial(
        pl.kernel, mesh=mesh,
        compiler_params=pltpu.CompilerParams(use_tc_tiling_on_sc=False),
        out_type=jax.ShapeDtypeStruct((NPAIR2, GC), jnp.float32),
        scratch_types=[
            pltpu.VMEM((2, CHUNK_PIX * 216), jnp.int32),
            pltpu.VMEM((2, CHUNK_PIX * 216 + 16), jnp.float32),
            pltpu.VMEM((2 * CHUNK_PIX * 216, GC), jnp.float32),
            pltpu.VMEM((CHUNK_PIX * GROUP, GC), jnp.float32),
            pltpu.SemaphoreType.DMA,
            pltpu.SemaphoreType.DMA,
            pltpu.SemaphoreType.DMA,
            pltpu.SemaphoreType.DMA,
        ],
    )
    def body(table_h, idx_h, wts_h, y_h, idx_v, wts_v, rows_v, out_v,
             sg0, sg1, si0, si1):
        wid = lax.axis_index("s") * 2 + lax.axis_index("c")
        sg = [sg0, sg1]
        si = [si0, si1]
        NV = CHUNK_PIX * 216

        def load_idx(t, b, sem):
            tc = jnp.minimum(t, NCHUNK - 1)
            pix0 = (wid * NCHUNK + tc) * CHUNK_PIX
            return pltpu.async_copy(idx_h.at[pl.ds(pix0 * 216, NV)],
                                    idx_v.at[b], sem)

        def load_wts(t, b, sem):
            tc = jnp.minimum(t, NCHUNK - 1)
            pix0 = (wid * NCHUNK + tc) * CHUNK_PIX
            return pltpu.async_copy(wts_h.at[pl.ds(pix0 * 216, NV)],
                                    wts_v.at[b, pl.ds(0, NV)], sem)

        def fire(b):
            return [
                pltpu.async_copy(
                    table_h.at[idx_v.at[b, pl.ds(j * 128, 128)]],
                    rows_v.at[pl.ds(b * NV + j * 128, 128)], sg[b])
                for j in range(IDX_ROWS)
            ]

        def compute(t, b):
            pix0 = (wid * NCHUNK + t) * CHUNK_PIX

            def per_pix(p, c2):
                base = p * 216
                accs = [jnp.zeros((GC,), jnp.float32) for _ in range(GROUP)]
                for c16 in range(14):
                    w16 = wts_v[b, pl.ds(base + c16 * 16, 16)]
                    nrow = 16 if c16 < 13 else 8
                    for tt in range(nrow):
                        j = c16 * 16 + tt
                        g = (j % 54) // KK
                        wv = jnp.full((GC,), w16[tt], jnp.float32)
                        accs[g] = accs[g] + wv * rows_v[b * NV + base + j, :]
                for g in range(GROUP):
                    out_v[p * GROUP + g, :] = accs[g]
                return c2

            lax.fori_loop(0, CHUNK_PIX, per_pix, 0)
            pltpu.sync_copy(out_v,
                            y_h.at[pl.ds(pix0 * GROUP, CHUNK_PIX * GROUP)])

        load_idx(0, 0, si[0]).wait()
        load_wts(0, 0, si[0]).wait()
        fire(0)
        load_idx(1, 1, si[1]).wait()
        load_wts(1, 1, si[1]).wait()

        def step(m, carry):
            for b in (0, 1):
                c = 2 * m + b

                @pl.when(c + 1 < NCHUNK)
                def _():
                    fire(1 - b)
                for j in range(IDX_ROWS):
                    pltpu.make_async_copy(
                        table_h.at[idx_v.at[b, pl.ds(j * 128, 128)]],
                        rows_v.at[pl.ds(b * NV + j * 128, 128)],
                        sg[b]).wait()
                p1 = load_idx(c + 2, b, si[b])
                compute(c, b)
                p2 = load_wts(c + 2, b, si[b])
                p1.wait()
                p2.wait()
            return carry

        lax.fori_loop(0, NCHUNK // 2, step, 0)

    return body(table, idx2, wts1)


def kernel(x, depth, W_in, b_in, W_dw, b_dw, ln_g, ln_b, W_off, b_off,
           W_mask, b_mask, W_out, b_out):
    wdw9 = W_dw.reshape(C_CH, KK).T
    woy = W_off[0::2]
    wox = W_off[1::2]
    wofft = jnp.concatenate([woy, wox], axis=0).T
    boffr = jnp.concatenate([b_off[0::2], b_off[1::2]]).reshape(1, -1)
    w_in_t = W_in.T
    w_out_t = W_out.T

    outs = []
    for n in range(N_B):
        xn = x[n:n + 1]
        x_proj = _matmul_bias(xn.reshape(NPIX2, C_CH), w_in_t, b_in)
        table = jnp.pad(
            x_proj.reshape(1, H_S, W_S, C_CH),
            ((0, 0), (PAD, PAD), (PAD, PAD), (0, 0))).reshape(NTAB2, GC)
        idx, wts = _prep(xn, wdw9, b_dw.reshape(1, -1), ln_g.reshape(1, -1),
                         ln_b.reshape(1, -1), wofft, boffr, W_mask.T,
                         b_mask.reshape(1, -1))
        y = _sc_sample(table, idx.reshape(-1), wts.reshape(-1))
        out = _matmul_bias(y.reshape(NPIX2, C_CH), w_out_t, b_out)
        outs.append(out.reshape(1, H_S, W_S, C_CH))
    return (jnp.concatenate(outs, axis=0), depth)


# speedup vs baseline: 32.3020x; 1.0599x over previous
import functools

import jax
import jax.numpy as jnp
from jax import lax
from jax.experimental import pallas as pl
from jax.experimental.pallas import tpu as pltpu
from jax.experimental.pallas import tpu_sc as plsc

N_B, H_S, W_S, C_CH = 2, 224, 224, 96
GROUP = 6
GC = C_CH // GROUP
KK = 9
PAD = 1
EPS = 1e-6
HP = H_S + 2 * PAD
NPIX = N_B * H_S * W_S
NPAIR = NPIX * GROUP
NTAB = N_B * HP * HP * GROUP

R_BLK = 16
NBLK = H_S // R_BLK
M_BLK = R_BLK * W_S

NW = 32
NPIX2 = NPIX // N_B
NPAIR2 = NPIX2 * GROUP
NTAB2 = HP * HP * GROUP
CHUNK_PIX = 16
NCHUNK = NPIX2 // NW // CHUNK_PIX
IDX_ROWS = (CHUNK_PIX * 216) // 128


def _matmul_body(y_ref, w_ref, b_ref, o_ref):
    o_ref[...] = (
        jnp.dot(y_ref[...], w_ref[...], preferred_element_type=jnp.float32)
        + b_ref[...]
    )


def _matmul_bias(y2d, w_t, b):
    m, c = y2d.shape
    blk = 1024 if m % 1024 == 0 else 896
    return pl.pallas_call(
        _matmul_body,
        grid=(m // blk,),
        in_specs=[
            pl.BlockSpec((blk, c), lambda i: (i, 0)),
            pl.BlockSpec((c, w_t.shape[1]), lambda i: (0, 0)),
            pl.BlockSpec((1, w_t.shape[1]), lambda i: (0, 0)),
        ],
        out_specs=pl.BlockSpec((blk, w_t.shape[1]), lambda i: (i, 0)),
        out_shape=jax.ShapeDtypeStruct((m, w_t.shape[1]), jnp.float32),
    )(y2d, w_t, b.reshape(1, -1))


def _erf(z):
    s = jnp.sign(z)
    za = jnp.abs(z)
    t = 1.0 / (1.0 + 0.3275911 * za)
    poly = t * (0.254829592 + t * (-0.284496736 + t * (1.421413741
               + t * (-1.453152027 + t * 1.061405429))))
    return s * (1.0 - poly * jnp.exp(-za * za))


def _prep_body(hb, xm_ref, xc_ref, xp_ref, wdw_ref, bdw_ref, lng_ref,
               lnb_ref, wofft_ref, boff_ref, wmaskt_ref, bmask_ref,
               idx_ref, wts_ref):
    n = pl.program_id(0)
    i = pl.program_id(1) + hb
    f32 = jnp.float32

    top = xm_ref[0, R_BLK - 1:R_BLK]
    bot = xp_ref[0, 0:1]
    xs = jnp.concatenate([top, xc_ref[0], bot], axis=0)
    ri = lax.broadcasted_iota(jnp.int32, (R_BLK + 2, 1, 1), 0)
    ok = ((ri != 0) | (i > 0)) & ((ri != R_BLK + 1) | (i < NBLK - 1))
    xs = xs * ok.astype(f32)

    acc = jnp.zeros((R_BLK, W_S, C_CH), f32)
    for ky in range(3):
        rows = xs[ky:ky + R_BLK]
        for kx in range(3):
            if kx == 0:
                sh = jnp.concatenate(
                    [jnp.zeros((R_BLK, 1, C_CH), f32), rows[:, :W_S - 1]], axis=1)
            elif kx == 1:
                sh = rows
            else:
                sh = jnp.concatenate(
                    [rows[:, 1:], jnp.zeros((R_BLK, 1, C_CH), f32)], axis=1)
            acc = acc + sh * wdw_ref[ky * 3 + kx]
    x1 = acc.reshape(M_BLK, C_CH) + bdw_ref[0]
    mu = jnp.mean(x1, axis=-1, keepdims=True)
    var = jnp.mean((x1 - mu) ** 2, axis=-1, keepdims=True)
    x1 = (x1 - mu) / jnp.sqrt(var + EPS) * lng_ref[0] + lnb_ref[0]
    x1 = x1 * 0.5 * (1.0 + _erf(x1 * 0.7071067811865476))

    offm = jnp.dot(x1, wofft_ref[...], preferred_element_type=f32) + boff_ref[0]
    ml = jnp.dot(x1, wmaskt_ref[...], preferred_element_type=f32) + bmask_ref[0]
    mx = jnp.max(ml, axis=-1, keepdims=True)
    e = jnp.exp(ml - mx)
    r54 = lax.broadcasted_iota(jnp.int32, (54, 54), 0)
    c54 = lax.broadcasted_iota(jnp.int32, (54, 54), 1)
    seg = ((r54 // KK) == (c54 // KK)).astype(f32)
    gs = jnp.dot(e, seg, preferred_element_type=f32)
    msk = e / gs

    mi = lax.broadcasted_iota(jnp.int32, (M_BLK, 1), 0)
    yb = (mi // W_S) + i * R_BLK + PAD
    xb = (mi % W_S) + PAD
    kk = lax.broadcasted_iota(jnp.int32, (1, 54), 1) % KK
    g54 = lax.broadcasted_iota(jnp.int32, (1, 54), 1) // KK
    dy = kk // 3 - 1
    dx = kk % 3 - 1
    py = yb.astype(f32) + dy.astype(f32) + offm[:, :54]
    px = xb.astype(f32) + dx.astype(f32) + offm[:, 54:]
    y0f = jnp.floor(py)
    x0f = jnp.floor(px)
    wy = py - y0f
    wx = px - x0f
    y0 = y0f.astype(jnp.int32)
    x0 = x0f.astype(jnp.int32)

    def cidx(iy, ix):
        v = ((iy >= 0) & (iy < HP) & (ix >= 0) & (ix < HP)).astype(f32)
        iyc = jnp.clip(iy, 0, HP - 1)
        ixc = jnp.clip(ix, 0, HP - 1)
        idx = ((n * HP + iyc) * HP + ixc) * GROUP + g54
        return idx, v

    i00, v00 = cidx(y0, x0)
    i01, v01 = cidx(y0, x0 + 1)
    i10, v10 = cidx(y0 + 1, x0)
    i11, v11 = cidx(y0 + 1, x0 + 1)
    wy1 = 1.0 - wy
    wx1 = 1.0 - wx
    idx_ref[:, 0:54] = i00
    idx_ref[:, 54:108] = i01
    idx_ref[:, 108:162] = i10
    idx_ref[:, 162:216] = i11
    wts_ref[:, 0:54] = wy1 * wx1 * msk * v00
    wts_ref[:, 54:108] = wy1 * wx * msk * v01
    wts_ref[:, 108:162] = wy * wx1 * msk * v10
    wts_ref[:, 162:216] = wy * wx * msk * v11


def _prep(x, wdw9, b_dw, ln_g, ln_b, wofft, boffr, wmaskt, b_mask,
          hb, nblk_h):
    xb = lambda d: pl.BlockSpec(
        (1, R_BLK, W_S, C_CH),
        lambda n, i, d=d: (n, jnp.clip(hb + i + d, 0, NBLK - 1), 0, 0))
    full = lambda a: pl.BlockSpec(a.shape, lambda n, i: (0,) * a.ndim)
    npix_h = nblk_h * M_BLK
    outs = [
        jax.ShapeDtypeStruct((npix_h, 216), jnp.int32),
        jax.ShapeDtypeStruct((npix_h, 216), jnp.float32),
    ]
    ospec = pl.BlockSpec((M_BLK, 216), lambda n, i: (i, 0))
    return pl.pallas_call(
        functools.partial(_prep_body, hb),
        grid=(1, nblk_h),
        in_specs=[xb(-1), xb(0), xb(1), full(wdw9), full(b_dw), full(ln_g),
                  full(ln_b), full(wofft), full(boffr), full(wmaskt),
                  full(b_mask)],
        out_specs=[ospec, ospec],
        out_shape=outs,
    )(x, x, x, wdw9, b_dw, ln_g, ln_b, wofft, boffr, wmaskt, b_mask)


def _sc_sample(table, idx2, wts1, npix):
    nchunk = npix // NW // CHUNK_PIX
    mesh = plsc.VectorSubcoreMesh(core_axis_name="c", subcore_axis_name="s")

    @functools.partial(
        pl.kernel, mesh=mesh,
        compiler_params=pltpu.CompilerParams(use_tc_tiling_on_sc=False),
        out_type=jax.ShapeDtypeStruct((npix * GROUP, GC), jnp.float32),
        scratch_types=[
            pltpu.VMEM((2, CHUNK_PIX * 216), jnp.int32),
            pltpu.VMEM((2, CHUNK_PIX * 216 + 16), jnp.float32),
            pltpu.VMEM((2 * CHUNK_PIX * 216, GC), jnp.float32),
            pltpu.VMEM((CHUNK_PIX * GROUP, GC), jnp.float32),
            pltpu.SemaphoreType.DMA,
            pltpu.SemaphoreType.DMA,
            pltpu.SemaphoreType.DMA,
            pltpu.SemaphoreType.DMA,
        ],
    )
    def body(table_h, idx_h, wts_h, y_h, idx_v, wts_v, rows_v, out_v,
             sg0, sg1, si0, si1):
        wid = lax.axis_index("s") * 2 + lax.axis_index("c")
        sg = [sg0, sg1]
        si = [si0, si1]
        NV = CHUNK_PIX * 216

        def load_idx(t, b, sem):
            tc = jnp.minimum(t, nchunk - 1)
            pix0 = (wid * nchunk + tc) * CHUNK_PIX
            return pltpu.async_copy(idx_h.at[pl.ds(pix0 * 216, NV)],
                                    idx_v.at[b], sem)

        def load_wts(t, b, sem):
            tc = jnp.minimum(t, nchunk - 1)
            pix0 = (wid * nchunk + tc) * CHUNK_PIX
            return pltpu.async_copy(wts_h.at[pl.ds(pix0 * 216, NV)],
                                    wts_v.at[b, pl.ds(0, NV)], sem)

        def fire(b):
            return [
                pltpu.async_copy(
                    table_h.at[idx_v.at[b, pl.ds(j * 128, 128)]],
                    rows_v.at[pl.ds(b * NV + j * 128, 128)], sg[b])
                for j in range(IDX_ROWS)
            ]

        def drain(b):
            pltpu.make_async_copy(
                table_h.at[idx_v.at[b]],
                rows_v.at[pl.ds(b * NV, NV)], sg[b]).wait()

        def compute(t, b):
            pix0 = (wid * nchunk + t) * CHUNK_PIX

            def per_pix(p, c2):
                base = p * 216
                accs = [jnp.zeros((GC,), jnp.float32) for _ in range(GROUP)]
                for c16 in range(14):
                    w16 = wts_v[b, pl.ds(base + c16 * 16, 16)]
                    nrow = 16 if c16 < 13 else 8
                    for tt in range(nrow):
                        j = c16 * 16 + tt
                        g = (j % 54) // KK
                        wv = jnp.full((GC,), w16[tt], jnp.float32)
                        accs[g] = accs[g] + wv * rows_v[b * NV + base + j, :]
                for g in range(GROUP):
                    out_v[p * GROUP + g, :] = accs[g]
                return c2

            lax.fori_loop(0, CHUNK_PIX, per_pix, 0)
            pltpu.sync_copy(out_v,
                            y_h.at[pl.ds(pix0 * GROUP, CHUNK_PIX * GROUP)])

        load_idx(0, 0, si[0]).wait()
        load_wts(0, 0, si[0]).wait()
        fire(0)
        load_idx(1, 1, si[1]).wait()
        load_wts(1, 1, si[1]).wait()

        def step(m, carry):
            for b in (0, 1):
                c = 2 * m + b

                @pl.when(c + 1 < nchunk)
                def _():
                    fire(1 - b)
                drain(b)
                p1 = load_idx(c + 2, b, si[b])
                compute(c, b)
                p2 = load_wts(c + 2, b, si[b])
                p1.wait()
                p2.wait()
            return carry

        lax.fori_loop(0, nchunk // 2, step, 0)
        if nchunk % 2:
            drain(0)
            compute(nchunk - 1, 0)

    return body(table, idx2, wts1)


def kernel(x, depth, W_in, b_in, W_dw, b_dw, ln_g, ln_b, W_off, b_off,
           W_mask, b_mask, W_out, b_out):
    wdw9 = W_dw.reshape(C_CH, KK).T
    woy = W_off[0::2]
    wox = W_off[1::2]
    wofft = jnp.concatenate([woy, wox], axis=0).T
    boffr = jnp.concatenate([b_off[0::2], b_off[1::2]]).reshape(1, -1)
    w_in_t = W_in.T
    w_out_t = W_out.T

    nblk_h = NBLK // 2
    npix_h = nblk_h * M_BLK
    outs = []
    for n in range(N_B):
        xn = x[n:n + 1]
        x_proj = _matmul_bias(xn.reshape(NPIX2, C_CH), w_in_t, b_in)
        table = jnp.pad(
            x_proj.reshape(1, H_S, W_S, C_CH),
            ((0, 0), (PAD, PAD), (PAD, PAD), (0, 0))).reshape(NTAB2, GC)
        for h in range(2):
            idx, wts = _prep(xn, wdw9, b_dw.reshape(1, -1),
                             ln_g.reshape(1, -1), ln_b.reshape(1, -1),
                             wofft, boffr, W_mask.T, b_mask.reshape(1, -1),
                             h * nblk_h, nblk_h)
            y = _sc_sample(table, idx.reshape(-1), wts.reshape(-1), npix_h)
            out = _matmul_bias(y.reshape(npix_h, C_CH), w_out_t, b_out)
            outs.append(out.reshape(1, nblk_h * R_BLK, W_S, C_CH))
    return (jnp.concatenate(
        [jnp.concatenate(outs[2 * n:2 * n + 2], axis=1) for n in range(N_B)],
        axis=0), depth)


# speedup vs baseline: 32.5194x; 1.0067x over previous
import functools

import jax
import jax.numpy as jnp
from jax import lax
from jax.experimental import pallas as pl
from jax.experimental.pallas import tpu as pltpu
from jax.experimental.pallas import tpu_sc as plsc

N_B, H_S, W_S, C_CH = 2, 224, 224, 96
GROUP = 6
GC = C_CH // GROUP
KK = 9
PAD = 1
EPS = 1e-6
HP = H_S + 2 * PAD
NPIX = N_B * H_S * W_S
NPAIR = NPIX * GROUP
NTAB = N_B * HP * HP * GROUP

R_BLK = 16
NBLK = H_S // R_BLK
M_BLK = R_BLK * W_S

NW = 32
NPIX2 = NPIX // N_B
NPAIR2 = NPIX2 * GROUP
NTAB2 = HP * HP * GROUP
CHUNK_PIX = 16
NCHUNK = NPIX2 // NW // CHUNK_PIX
IDX_ROWS = (CHUNK_PIX * 216) // 128


def _matmul_body(y_ref, w_ref, b_ref, o_ref):
    o_ref[...] = (
        jnp.dot(y_ref[...], w_ref[...], preferred_element_type=jnp.float32)
        + b_ref[...]
    )


def _matmul_bias(y2d, w_t, b):
    m, c = y2d.shape
    blk = 1024 if m % 1024 == 0 else 896
    return pl.pallas_call(
        _matmul_body,
        grid=(m // blk,),
        in_specs=[
            pl.BlockSpec((blk, c), lambda i: (i, 0)),
            pl.BlockSpec((c, w_t.shape[1]), lambda i: (0, 0)),
            pl.BlockSpec((1, w_t.shape[1]), lambda i: (0, 0)),
        ],
        out_specs=pl.BlockSpec((blk, w_t.shape[1]), lambda i: (i, 0)),
        out_shape=jax.ShapeDtypeStruct((m, w_t.shape[1]), jnp.float32),
    )(y2d, w_t, b.reshape(1, -1))


def _erf(z):
    s = jnp.sign(z)
    za = jnp.abs(z)
    t = 1.0 / (1.0 + 0.3275911 * za)
    poly = t * (0.254829592 + t * (-0.284496736 + t * (1.421413741
               + t * (-1.453152027 + t * 1.061405429))))
    return s * (1.0 - poly * jnp.exp(-za * za))


def _prep_body(hb, xm_ref, xc_ref, xp_ref, wdw_ref, bdw_ref, lng_ref,
               lnb_ref, wofft_ref, boff_ref, wmaskt_ref, bmask_ref,
               idx_ref, wts_ref):
    n = pl.program_id(0)
    i = pl.program_id(1) + hb
    f32 = jnp.float32

    top = xm_ref[0, R_BLK - 1:R_BLK]
    bot = xp_ref[0, 0:1]
    xs = jnp.concatenate([top, xc_ref[0], bot], axis=0)
    ri = lax.broadcasted_iota(jnp.int32, (R_BLK + 2, 1, 1), 0)
    ok = ((ri != 0) | (i > 0)) & ((ri != R_BLK + 1) | (i < NBLK - 1))
    xs = xs * ok.astype(f32)

    acc = jnp.zeros((R_BLK, W_S, C_CH), f32)
    for ky in range(3):
        rows = xs[ky:ky + R_BLK]
        for kx in range(3):
            if kx == 0:
                sh = jnp.concatenate(
                    [jnp.zeros((R_BLK, 1, C_CH), f32), rows[:, :W_S - 1]], axis=1)
            elif kx == 1:
                sh = rows
            else:
                sh = jnp.concatenate(
                    [rows[:, 1:], jnp.zeros((R_BLK, 1, C_CH), f32)], axis=1)
            acc = acc + sh * wdw_ref[ky * 3 + kx]
    x1 = acc.reshape(M_BLK, C_CH) + bdw_ref[0]
    mu = jnp.mean(x1, axis=-1, keepdims=True)
    var = jnp.mean((x1 - mu) ** 2, axis=-1, keepdims=True)
    x1 = (x1 - mu) / jnp.sqrt(var + EPS) * lng_ref[0] + lnb_ref[0]
    x1 = x1 * 0.5 * (1.0 + _erf(x1 * 0.7071067811865476))

    offm = jnp.dot(x1, wofft_ref[...], preferred_element_type=f32) + boff_ref[0]
    ml = jnp.dot(x1, wmaskt_ref[...], preferred_element_type=f32) + bmask_ref[0]
    mx = jnp.max(ml, axis=-1, keepdims=True)
    e = jnp.exp(ml - mx)
    r54 = lax.broadcasted_iota(jnp.int32, (54, 54), 0)
    c54 = lax.broadcasted_iota(jnp.int32, (54, 54), 1)
    seg = ((r54 // KK) == (c54 // KK)).astype(f32)
    gs = jnp.dot(e, seg, preferred_element_type=f32)
    msk = e / gs

    mi = lax.broadcasted_iota(jnp.int32, (M_BLK, 1), 0)
    yb = (mi // W_S) + i * R_BLK + PAD
    xb = (mi % W_S) + PAD
    kk = lax.broadcasted_iota(jnp.int32, (1, 54), 1) % KK
    g54 = lax.broadcasted_iota(jnp.int32, (1, 54), 1) // KK
    dy = kk // 3 - 1
    dx = kk % 3 - 1
    py = yb.astype(f32) + dy.astype(f32) + offm[:, :54]
    px = xb.astype(f32) + dx.astype(f32) + offm[:, 54:]
    y0f = jnp.floor(py)
    x0f = jnp.floor(px)
    wy = py - y0f
    wx = px - x0f
    y0 = y0f.astype(jnp.int32)
    x0 = x0f.astype(jnp.int32)

    def cidx(iy, ix):
        v = ((iy >= 1) & (iy < HP - 1) & (ix >= 1) & (ix < HP - 1)
             ).astype(f32)
        iyc = jnp.clip(iy - 1, 0, H_S - 1)
        ixc = jnp.clip(ix - 1, 0, W_S - 1)
        idx = (iyc * W_S + ixc) * GROUP + g54
        return idx, v

    i00, v00 = cidx(y0, x0)
    i01, v01 = cidx(y0, x0 + 1)
    i10, v10 = cidx(y0 + 1, x0)
    i11, v11 = cidx(y0 + 1, x0 + 1)
    wy1 = 1.0 - wy
    wx1 = 1.0 - wx
    idx_ref[:, 0:54] = i00
    idx_ref[:, 54:108] = i01
    idx_ref[:, 108:162] = i10
    idx_ref[:, 162:216] = i11
    wts_ref[:, 0:54] = wy1 * wx1 * msk * v00
    wts_ref[:, 54:108] = wy1 * wx * msk * v01
    wts_ref[:, 108:162] = wy * wx1 * msk * v10
    wts_ref[:, 162:216] = wy * wx * msk * v11


def _prep(x, wdw9, b_dw, ln_g, ln_b, wofft, boffr, wmaskt, b_mask,
          hb, nblk_h):
    xb = lambda d: pl.BlockSpec(
        (1, R_BLK, W_S, C_CH),
        lambda n, i, d=d: (n, jnp.clip(hb + i + d, 0, NBLK - 1), 0, 0))
    full = lambda a: pl.BlockSpec(a.shape, lambda n, i: (0,) * a.ndim)
    npix_h = nblk_h * M_BLK
    outs = [
        jax.ShapeDtypeStruct((npix_h, 216), jnp.int32),
        jax.ShapeDtypeStruct((npix_h, 216), jnp.float32),
    ]
    ospec = pl.BlockSpec((M_BLK, 216), lambda n, i: (i, 0))
    return pl.pallas_call(
        functools.partial(_prep_body, hb),
        grid=(1, nblk_h),
        in_specs=[xb(-1), xb(0), xb(1), full(wdw9), full(b_dw), full(ln_g),
                  full(ln_b), full(wofft), full(boffr), full(wmaskt),
                  full(b_mask)],
        out_specs=[ospec, ospec],
        out_shape=outs,
    )(x, x, x, wdw9, b_dw, ln_g, ln_b, wofft, boffr, wmaskt, b_mask)


def _sc_sample(table, idx2, wts1, npix):
    nchunk = npix // NW // CHUNK_PIX
    mesh = plsc.VectorSubcoreMesh(core_axis_name="c", subcore_axis_name="s")

    @functools.partial(
        pl.kernel, mesh=mesh,
        compiler_params=pltpu.CompilerParams(use_tc_tiling_on_sc=False),
        out_type=jax.ShapeDtypeStruct((npix * GROUP, GC), jnp.float32),
        scratch_types=[
            pltpu.VMEM((2, CHUNK_PIX * 216), jnp.int32),
            pltpu.VMEM((2, CHUNK_PIX * 216 + 16), jnp.float32),
            pltpu.VMEM((2 * CHUNK_PIX * 216, GC), jnp.float32),
            pltpu.VMEM((CHUNK_PIX * GROUP, GC), jnp.float32),
            pltpu.SemaphoreType.DMA,
            pltpu.SemaphoreType.DMA,
            pltpu.SemaphoreType.DMA,
            pltpu.SemaphoreType.DMA,
        ],
    )
    def body(table_h, idx_h, wts_h, y_h, idx_v, wts_v, rows_v, out_v,
             sg0, sg1, si0, si1):
        wid = lax.axis_index("s") * 2 + lax.axis_index("c")
        sg = [sg0, sg1]
        si = [si0, si1]
        NV = CHUNK_PIX * 216

        def load_idx(t, b, sem):
            tc = jnp.minimum(t, nchunk - 1)
            pix0 = (wid * nchunk + tc) * CHUNK_PIX
            return pltpu.async_copy(idx_h.at[pl.ds(pix0 * 216, NV)],
                                    idx_v.at[b], sem)

        def load_wts(t, b, sem):
            tc = jnp.minimum(t, nchunk - 1)
            pix0 = (wid * nchunk + tc) * CHUNK_PIX
            return pltpu.async_copy(wts_h.at[pl.ds(pix0 * 216, NV)],
                                    wts_v.at[b, pl.ds(0, NV)], sem)

        def fire(b):
            return [
                pltpu.async_copy(
                    table_h.at[idx_v.at[b, pl.ds(j * 128, 128)]],
                    rows_v.at[pl.ds(b * NV + j * 128, 128)], sg[b])
                for j in range(IDX_ROWS)
            ]

        def drain(b):
            pltpu.make_async_copy(
                table_h.at[idx_v.at[b]],
                rows_v.at[pl.ds(b * NV, NV)], sg[b]).wait()

        def compute(t, b):
            pix0 = (wid * nchunk + t) * CHUNK_PIX

            def per_pix(p, c2):
                base = p * 216
                accs = [jnp.zeros((GC,), jnp.float32) for _ in range(GROUP)]
                for c16 in range(14):
                    w16 = wts_v[b, pl.ds(base + c16 * 16, 16)]
                    nrow = 16 if c16 < 13 else 8
                    for tt in range(nrow):
                        j = c16 * 16 + tt
                        g = (j % 54) // KK
                        wv = jnp.full((GC,), w16[tt], jnp.float32)
                        accs[g] = accs[g] + wv * rows_v[b * NV + base + j, :]
                for g in range(GROUP):
                    out_v[p * GROUP + g, :] = accs[g]
                return c2

            lax.fori_loop(0, CHUNK_PIX, per_pix, 0)
            pltpu.sync_copy(out_v,
                            y_h.at[pl.ds(pix0 * GROUP, CHUNK_PIX * GROUP)])

        load_idx(0, 0, si[0]).wait()
        load_wts(0, 0, si[0]).wait()
        fire(0)
        load_idx(1, 1, si[1]).wait()
        load_wts(1, 1, si[1]).wait()

        def step(m, carry):
            for b in (0, 1):
                c = 2 * m + b

                @pl.when(c + 1 < nchunk)
                def _():
                    fire(1 - b)
                drain(b)
                p1 = load_idx(c + 2, b, si[b])
                compute(c, b)
                p2 = load_wts(c + 2, b, si[b])
                p1.wait()
                p2.wait()
            return carry

        lax.fori_loop(0, nchunk // 2, step, 0)
        if nchunk % 2:
            drain(0)
            compute(nchunk - 1, 0)

    return body(table, idx2, wts1)


def kernel(x, depth, W_in, b_in, W_dw, b_dw, ln_g, ln_b, W_off, b_off,
           W_mask, b_mask, W_out, b_out):
    wdw9 = W_dw.reshape(C_CH, KK).T
    woy = W_off[0::2]
    wox = W_off[1::2]
    wofft = jnp.concatenate([woy, wox], axis=0).T
    boffr = jnp.concatenate([b_off[0::2], b_off[1::2]]).reshape(1, -1)
    w_in_t = W_in.T
    w_out_t = W_out.T

    nblk_h = NBLK // 2
    npix_h = nblk_h * M_BLK
    outs = []
    for n in range(N_B):
        xn = x[n:n + 1]
        x_proj = _matmul_bias(xn.reshape(NPIX2, C_CH), w_in_t, b_in)
        table = x_proj.reshape(NPIX2 * GROUP, GC)
        for h in range(2):
            idx, wts = _prep(xn, wdw9, b_dw.reshape(1, -1),
                             ln_g.reshape(1, -1), ln_b.reshape(1, -1),
                             wofft, boffr, W_mask.T, b_mask.reshape(1, -1),
                             h * nblk_h, nblk_h)
            y = _sc_sample(table, idx.reshape(-1), wts.reshape(-1), npix_h)
            out = _matmul_bias(y.reshape(npix_h, C_CH), w_out_t, b_out)
            outs.append(out.reshape(1, nblk_h * R_BLK, W_S, C_CH))
    return (jnp.concatenate(
        [jnp.concatenate(outs[2 * n:2 * n + 2], axis=1) for n in range(N_B)],
        axis=0), depth)


# speedup vs baseline: 33.2703x; 1.0231x over previous
import functools

import jax
import jax.numpy as jnp
from jax import lax
from jax.experimental import pallas as pl
from jax.experimental.pallas import tpu as pltpu
from jax.experimental.pallas import tpu_sc as plsc

N_B, H_S, W_S, C_CH = 2, 224, 224, 96
GROUP = 6
GC = C_CH // GROUP
KK = 9
PAD = 1
EPS = 1e-6
HP = H_S + 2 * PAD
NPIX = N_B * H_S * W_S
NPAIR = NPIX * GROUP
NTAB = N_B * HP * HP * GROUP

R_BLK = 16
NBLK = H_S // R_BLK
M_BLK = R_BLK * W_S

NW = 32
NPIX2 = NPIX // N_B
NPAIR2 = NPIX2 * GROUP
NTAB2 = HP * HP * GROUP
CHUNK_PIX = 16
NCHUNK = NPIX2 // NW // CHUNK_PIX
IDX_ROWS = (CHUNK_PIX * 216) // 128


def _matmul_body(y_ref, w_ref, b_ref, o_ref):
    o_ref[...] = (
        jnp.dot(y_ref[...], w_ref[...], preferred_element_type=jnp.float32)
        + b_ref[...]
    )


def _matmul_bias(y2d, w_t, b):
    m, c = y2d.shape
    blk = 1024 if m % 1024 == 0 else 896
    return pl.pallas_call(
        _matmul_body,
        grid=(m // blk,),
        in_specs=[
            pl.BlockSpec((blk, c), lambda i: (i, 0)),
            pl.BlockSpec((c, w_t.shape[1]), lambda i: (0, 0)),
            pl.BlockSpec((1, w_t.shape[1]), lambda i: (0, 0)),
        ],
        out_specs=pl.BlockSpec((blk, w_t.shape[1]), lambda i: (i, 0)),
        out_shape=jax.ShapeDtypeStruct((m, w_t.shape[1]), jnp.float32),
    )(y2d, w_t, b.reshape(1, -1))


def _erf(z):
    s = jnp.sign(z)
    za = jnp.abs(z)
    t = 1.0 / (1.0 + 0.3275911 * za)
    poly = t * (0.254829592 + t * (-0.284496736 + t * (1.421413741
               + t * (-1.453152027 + t * 1.061405429))))
    return s * (1.0 - poly * jnp.exp(-za * za))


def _prep_body(hb, xm_ref, xc_ref, xp_ref, wdw_ref, bdw_ref, lng_ref,
               lnb_ref, wofft_ref, boff_ref, wmaskt_ref, bmask_ref,
               idx_ref, wts_ref):
    n = pl.program_id(0)
    i = pl.program_id(1) + hb
    f32 = jnp.float32

    top = xm_ref[0, R_BLK - 1:R_BLK]
    bot = xp_ref[0, 0:1]
    xs = jnp.concatenate([top, xc_ref[0], bot], axis=0)
    ri = lax.broadcasted_iota(jnp.int32, (R_BLK + 2, 1, 1), 0)
    ok = ((ri != 0) | (i > 0)) & ((ri != R_BLK + 1) | (i < NBLK - 1))
    xs = xs * ok.astype(f32)

    acc = jnp.zeros((R_BLK, W_S, C_CH), f32)
    for ky in range(3):
        rows = xs[ky:ky + R_BLK]
        for kx in range(3):
            if kx == 0:
                sh = jnp.concatenate(
                    [jnp.zeros((R_BLK, 1, C_CH), f32), rows[:, :W_S - 1]], axis=1)
            elif kx == 1:
                sh = rows
            else:
                sh = jnp.concatenate(
                    [rows[:, 1:], jnp.zeros((R_BLK, 1, C_CH), f32)], axis=1)
            acc = acc + sh * wdw_ref[ky * 3 + kx]
    x1 = acc.reshape(M_BLK, C_CH) + bdw_ref[0]
    mu = jnp.mean(x1, axis=-1, keepdims=True)
    var = jnp.mean((x1 - mu) ** 2, axis=-1, keepdims=True)
    x1 = (x1 - mu) / jnp.sqrt(var + EPS) * lng_ref[0] + lnb_ref[0]
    x1 = x1 * 0.5 * (1.0 + _erf(x1 * 0.7071067811865476))

    offm = jnp.dot(x1, wofft_ref[...], preferred_element_type=f32) + boff_ref[0]
    ml = jnp.dot(x1, wmaskt_ref[...], preferred_element_type=f32) + bmask_ref[0]
    mx = jnp.max(ml, axis=-1, keepdims=True)
    e = jnp.exp(ml - mx)
    r54 = lax.broadcasted_iota(jnp.int32, (54, 54), 0)
    c54 = lax.broadcasted_iota(jnp.int32, (54, 54), 1)
    seg = ((r54 // KK) == (c54 // KK)).astype(f32)
    gs = jnp.dot(e, seg, preferred_element_type=f32)
    msk = e / gs

    mi = lax.broadcasted_iota(jnp.int32, (M_BLK, 1), 0)
    yb = (mi // W_S) + i * R_BLK + PAD
    xb = (mi % W_S) + PAD
    kk = lax.broadcasted_iota(jnp.int32, (1, 54), 1) % KK
    g54 = lax.broadcasted_iota(jnp.int32, (1, 54), 1) // KK
    dy = kk // 3 - 1
    dx = kk % 3 - 1
    py = yb.astype(f32) + dy.astype(f32) + offm[:, :54]
    px = xb.astype(f32) + dx.astype(f32) + offm[:, 54:]
    y0f = jnp.floor(py)
    x0f = jnp.floor(px)
    wy = py - y0f
    wx = px - x0f
    y0 = y0f.astype(jnp.int32)
    x0 = x0f.astype(jnp.int32)

    def cidx(iy, ix):
        v = ((iy >= 1) & (iy < HP - 1) & (ix >= 1) & (ix < HP - 1)
             ).astype(f32)
        iyc = jnp.clip(iy - 1, 0, H_S - 1)
        ixc = jnp.clip(ix - 1, 0, W_S - 1)
        idx = (iyc * W_S + ixc) * GROUP + g54
        return idx, v

    i00, v00 = cidx(y0, x0)
    i01, v01 = cidx(y0, x0 + 1)
    i10, v10 = cidx(y0 + 1, x0)
    i11, v11 = cidx(y0 + 1, x0 + 1)
    wy1 = 1.0 - wy
    wx1 = 1.0 - wx
    idx_ref[:, 0:54] = i00
    idx_ref[:, 54:108] = i01
    idx_ref[:, 108:162] = i10
    idx_ref[:, 162:216] = i11
    wts_ref[:, 0:54] = wy1 * wx1 * msk * v00
    wts_ref[:, 54:108] = wy1 * wx * msk * v01
    wts_ref[:, 108:162] = wy * wx1 * msk * v10
    wts_ref[:, 162:216] = wy * wx * msk * v11


def _prep(x, wdw9, b_dw, ln_g, ln_b, wofft, boffr, wmaskt, b_mask,
          hb, nblk_h):
    xb = lambda d: pl.BlockSpec(
        (1, R_BLK, W_S, C_CH),
        lambda n, i, d=d: (n, jnp.clip(hb + i + d, 0, NBLK - 1), 0, 0))
    full = lambda a: pl.BlockSpec(a.shape, lambda n, i: (0,) * a.ndim)
    npix_h = nblk_h * M_BLK
    outs = [
        jax.ShapeDtypeStruct((npix_h, 216), jnp.int32),
        jax.ShapeDtypeStruct((npix_h, 216), jnp.float32),
    ]
    ospec = pl.BlockSpec((M_BLK, 216), lambda n, i: (i, 0))
    return pl.pallas_call(
        functools.partial(_prep_body, hb),
        grid=(1, nblk_h),
        in_specs=[xb(-1), xb(0), xb(1), full(wdw9), full(b_dw), full(ln_g),
                  full(ln_b), full(wofft), full(boffr), full(wmaskt),
                  full(b_mask)],
        out_specs=[ospec, ospec],
        out_shape=outs,
    )(x, x, x, wdw9, b_dw, ln_g, ln_b, wofft, boffr, wmaskt, b_mask)


def _sc_sample(table, idx2, wts1, npix):
    nchunk = npix // NW // CHUNK_PIX
    mesh = plsc.VectorSubcoreMesh(core_axis_name="c", subcore_axis_name="s")

    @functools.partial(
        pl.kernel, mesh=mesh,
        compiler_params=pltpu.CompilerParams(use_tc_tiling_on_sc=False),
        out_type=jax.ShapeDtypeStruct((npix * GROUP, GC), jnp.float32),
        scratch_types=[
            pltpu.VMEM((2, CHUNK_PIX * 216), jnp.int32),
            pltpu.VMEM((2, CHUNK_PIX * 216 + 16), jnp.float32),
            pltpu.VMEM((2 * CHUNK_PIX * 216, GC), jnp.float32),
            pltpu.VMEM((CHUNK_PIX * GROUP, GC), jnp.float32),
            pltpu.SemaphoreType.DMA,
            pltpu.SemaphoreType.DMA,
            pltpu.SemaphoreType.DMA,
            pltpu.SemaphoreType.DMA,
        ],
    )
    def body(table_h, idx_h, wts_h, y_h, idx_v, wts_v, rows_v, out_v,
             sg0, sg1, si0, si1):
        wid = lax.axis_index("s") * 2 + lax.axis_index("c")
        sg = [sg0, sg1]
        si = [si0, si1]
        NV = CHUNK_PIX * 216

        def load_idx(t, b, sem):
            tc = jnp.minimum(t, nchunk - 1)
            pix0 = (wid * nchunk + tc) * CHUNK_PIX
            return pltpu.async_copy(idx_h.at[pl.ds(pix0 * 216, NV)],
                                    idx_v.at[b], sem)

        def load_wts(t, b, sem):
            tc = jnp.minimum(t, nchunk - 1)
            pix0 = (wid * nchunk + tc) * CHUNK_PIX
            return pltpu.async_copy(wts_h.at[pl.ds(pix0 * 216, NV)],
                                    wts_v.at[b, pl.ds(0, NV)], sem)

        def fire(b):
            return [
                pltpu.async_copy(
                    table_h.at[idx_v.at[b, pl.ds(j * 128, 128)]],
                    rows_v.at[pl.ds(b * NV + j * 128, 128)], sg[b])
                for j in range(IDX_ROWS)
            ]

        def drain(b):
            pltpu.make_async_copy(
                table_h.at[idx_v.at[b]],
                rows_v.at[pl.ds(b * NV, NV)], sg[b]).wait()

        def compute(t, b):
            pix0 = (wid * nchunk + t) * CHUNK_PIX

            def per_pix(p, c2):
                base = p * 216
                accs = [jnp.zeros((GC,), jnp.float32) for _ in range(GROUP)]
                for c16 in range(14):
                    w16 = wts_v[b, pl.ds(base + c16 * 16, 16)]
                    nrow = 16 if c16 < 13 else 8
                    for tt in range(nrow):
                        j = c16 * 16 + tt
                        g = (j % 54) // KK
                        wv = jnp.full((GC,), w16[tt], jnp.float32)
                        accs[g] = accs[g] + wv * rows_v[b * NV + base + j, :]
                for g in range(GROUP):
                    out_v[p * GROUP + g, :] = accs[g]
                return c2

            lax.fori_loop(0, CHUNK_PIX, per_pix, 0)
            pltpu.sync_copy(out_v,
                            y_h.at[pl.ds(pix0 * GROUP, CHUNK_PIX * GROUP)])

        def wait_wts(b):
            pltpu.make_async_copy(wts_h.at[pl.ds(0, NV)],
                                  wts_v.at[b, pl.ds(0, NV)], si[b]).wait()

        load_idx(0, 0, si[0]).wait()
        load_wts(0, 0, si[0])
        fire(0)
        load_idx(1, 1, si[1]).wait()
        load_wts(1, 1, si[1])

        def step(m, carry):
            for b in (0, 1):
                c = 2 * m + b

                @pl.when(c + 1 < nchunk)
                def _():
                    fire(1 - b)
                drain(b)
                p1 = load_idx(c + 2, b, si[b])
                wait_wts(b)
                compute(c, b)
                load_wts(c + 2, b, si[b])
                p1.wait()
            return carry

        lax.fori_loop(0, nchunk // 2, step, 0)
        if nchunk % 2:
            drain(0)
            wait_wts(0)
            compute(nchunk - 1, 0)
            wait_wts(1)
        else:
            wait_wts(0)
            wait_wts(1)

    return body(table, idx2, wts1)


def kernel(x, depth, W_in, b_in, W_dw, b_dw, ln_g, ln_b, W_off, b_off,
           W_mask, b_mask, W_out, b_out):
    wdw9 = W_dw.reshape(C_CH, KK).T
    woy = W_off[0::2]
    wox = W_off[1::2]
    wofft = jnp.concatenate([woy, wox], axis=0).T
    boffr = jnp.concatenate([b_off[0::2], b_off[1::2]]).reshape(1, -1)
    w_in_t = W_in.T
    w_out_t = W_out.T

    nblk_h = NBLK // 2
    npix_h = nblk_h * M_BLK
    outs = []
    for n in range(N_B):
        xn = x[n:n + 1]
        x_proj = _matmul_bias(xn.reshape(NPIX2, C_CH), w_in_t, b_in)
        table = x_proj.reshape(NPIX2 * GROUP, GC)
        for h in range(2):
            idx, wts = _prep(xn, wdw9, b_dw.reshape(1, -1),
                             ln_g.reshape(1, -1), ln_b.reshape(1, -1),
                             wofft, boffr, W_mask.T, b_mask.reshape(1, -1),
                             h * nblk_h, nblk_h)
            y = _sc_sample(table, idx.reshape(-1), wts.reshape(-1), npix_h)
            out = _matmul_bias(y.reshape(npix_h, C_CH), w_out_t, b_out)
            outs.append(out.reshape(1, nblk_h * R_BLK, W_S, C_CH))
    return (jnp.concatenate(
        [jnp.concatenate(outs[2 * n:2 * n + 2], axis=1) for n in range(N_B)],
        axis=0), depth)
